# Initial kernel scaffold; baseline (speedup 1.0000x reference)
import jax
import jax.numpy as jnp
from jax.experimental import pallas as pl


def kernel(simmat, dtoks, qtoks):
    raise NotImplementedError("write your pallas kernel here")


# speedup vs baseline: 30.4224x; 30.4224x over previous
import functools

import jax
import jax.numpy as jnp
from jax import lax
from jax.experimental import pallas as pl
from jax.experimental.pallas import tpu as pltpu
from jax.experimental.pallas import tpu_sc as plsc

_BINS = 30
_HBINS = 32
_L = 16
_NC = 2
_NS = 16
_NW = _NC * _NS


def _sc_hist(simmat, dtoks):
    B, Q, D = simmat.shape
    b_per_w = B // _NW
    nchunk = D // _L

    mesh = plsc.VectorSubcoreMesh(core_axis_name="c", subcore_axis_name="s")

    @functools.partial(
        pl.kernel,
        mesh=mesh,
        compiler_params=pltpu.CompilerParams(needs_layout_passes=False),
        out_type=jax.ShapeDtypeStruct((B, Q * _HBINS), jnp.float32),
        scratch_types=[
            pltpu.VMEM((D,), jnp.int32),
            pltpu.VMEM((D,), jnp.float32),
            pltpu.VMEM((2, D), jnp.float32),
            pltpu.VMEM((_L * _HBINS,), jnp.float32),
            pltpu.VMEM((Q * _HBINS,), jnp.float32),
            pltpu.SemaphoreType.DMA,
            pltpu.SemaphoreType.DMA,
        ],
    )
    def hist_kernel(simmat_hbm, dtoks_hbm, out_hbm,
                    dbuf, wbuf, sbuf, hbuf, obuf, sem_s, sem_d):
        wid = lax.axis_index("s") * _NC + lax.axis_index("c")
        lane_off = lax.iota(jnp.int32, _L) * _HBINS

        for bi in range(b_per_w):
            b = wid * b_per_w + bi

            pltpu.async_copy(dtoks_hbm.at[b], dbuf, sem_d).wait()

            def wbody(i, _):
                t = dbuf[pl.ds(i * _L, _L)]
                wbuf[pl.ds(i * _L, _L)] = jnp.where(
                    t != 0,
                    jnp.full((_L,), 1.0, jnp.float32),
                    jnp.full((_L,), 0.0, jnp.float32))
                return 0

            lax.fori_loop(0, nchunk, wbody, 0)

            copies = [None, None]
            copies[0] = pltpu.async_copy(simmat_hbm.at[b, 0], sbuf.at[0],
                                         sem_s)
            for q in range(Q):
                cur = q % 2
                copies[cur].wait()
                if q + 1 < Q:
                    copies[1 - cur] = pltpu.async_copy(
                        simmat_hbm.at[b, q + 1], sbuf.at[1 - cur], sem_s)

                for k in range(_HBINS):
                    hbuf[pl.ds(k * _L, _L)] = jnp.zeros((_L,), jnp.float32)

                def cbody(i, _, cur=cur):
                    v = sbuf[cur, pl.ds(i * _L, _L)]
                    t = (v + jnp.float32(1.000001)) * jnp.float32(0.5)
                    bins = (t * jnp.float32(_BINS - 1)).astype(jnp.int32)
                    w = wbuf[pl.ds(i * _L, _L)]
                    plsc.addupdate_scatter(hbuf, [bins + lane_off], w)
                    return 0

                lax.fori_loop(0, nchunk, cbody, 0)

                for h in range(_HBINS // _L):
                    acc = hbuf[pl.ds(h * _L, _L)]
                    for l in range(1, _L):
                        acc = acc + hbuf[pl.ds(l * _HBINS + h * _L, _L)]
                    obuf[pl.ds(q * _HBINS + h * _L, _L)] = acc

            pltpu.async_copy(obuf, out_hbm.at[b], sem_d).wait()

    return hist_kernel(simmat, dtoks)


def _tc_finish(hist2, qtoks):
    BQ = hist2.shape[0]
    q2 = qtoks.reshape(BQ, 1)

    def body(h_ref, q_ref, o_ref):
        qm = (q_ref[...] != 0).astype(jnp.float32)
        o_ref[...] = jnp.log(h_ref[...] * qm + jnp.float32(1e-5))

    return pl.pallas_call(
        body,
        out_shape=jax.ShapeDtypeStruct((BQ, _HBINS), jnp.float32),
    )(hist2, q2)


def kernel(simmat, dtoks, qtoks):
    B, Q, _ = simmat.shape
    hist = _sc_hist(simmat, dtoks.astype(jnp.int32))
    hist2 = hist.reshape(B * Q, _HBINS)
    out = _tc_finish(hist2, qtoks.astype(jnp.int32))
    return out[:, :_BINS].reshape(B, Q, _BINS)


# speedup vs baseline: 33.4147x; 1.0984x over previous
import functools

import jax
import jax.numpy as jnp
from jax import lax
from jax.experimental import pallas as pl
from jax.experimental.pallas import tpu as pltpu
from jax.experimental.pallas import tpu_sc as plsc

_BINS = 30
_HBINS = 32
_L = 16
_NC = 2
_NS = 16
_NW = _NC * _NS


def _sc_hist(simmat, dtoks):
    B, Q, D = simmat.shape
    b_per_w = B // _NW
    nchunk = D // _L

    mesh = plsc.VectorSubcoreMesh(core_axis_name="c", subcore_axis_name="s")

    @functools.partial(
        pl.kernel,
        mesh=mesh,
        compiler_params=pltpu.CompilerParams(needs_layout_passes=False),
        out_type=jax.ShapeDtypeStruct((B, Q * _HBINS), jnp.float32),
        scratch_types=[
            pltpu.VMEM((D,), jnp.int32),
            pltpu.VMEM((D,), jnp.float32),
            pltpu.VMEM((2, D), jnp.float32),
            pltpu.VMEM((_L * _HBINS,), jnp.float32),
            pltpu.VMEM((Q * _HBINS,), jnp.float32),
            pltpu.SemaphoreType.DMA,
            pltpu.SemaphoreType.DMA,
        ],
    )
    def hist_kernel(simmat_hbm, dtoks_hbm, out_hbm,
                    dbuf, wbuf, sbuf, hbuf, obuf, sem_s, sem_d):
        wid = lax.axis_index("s") * _NC + lax.axis_index("c")
        lane_off = lax.iota(jnp.int32, _L) * _HBINS

        for bi in range(b_per_w):
            b = wid * b_per_w + bi

            pltpu.async_copy(dtoks_hbm.at[b], dbuf, sem_d).wait()

            def wbody(i, _):
                for u in range(4):
                    t = dbuf[pl.ds((i * 4 + u) * _L, _L)]
                    wbuf[pl.ds((i * 4 + u) * _L, _L)] = jnp.where(
                        t != 0,
                        jnp.full((_L,), 1.0, jnp.float32),
                        jnp.full((_L,), 0.0, jnp.float32))
                return 0

            lax.fori_loop(0, nchunk // 4, wbody, 0)

            copies = [None, None]
            copies[0] = pltpu.async_copy(simmat_hbm.at[b, 0], sbuf.at[0],
                                         sem_s)
            for q in range(Q):
                cur = q % 2
                copies[cur].wait()
                if q + 1 < Q:
                    copies[1 - cur] = pltpu.async_copy(
                        simmat_hbm.at[b, q + 1], sbuf.at[1 - cur], sem_s)

                for k in range(_HBINS):
                    hbuf[pl.ds(k * _L, _L)] = jnp.zeros((_L,), jnp.float32)

                def cbody(i, _, cur=cur):
                    for u in range(8):
                        j = i * 8 + u
                        v = sbuf[cur, pl.ds(j * _L, _L)]
                        t = (v + jnp.float32(1.000001)) * jnp.float32(0.5)
                        bins = (t * jnp.float32(_BINS - 1)).astype(jnp.int32)
                        w = wbuf[pl.ds(j * _L, _L)]
                        plsc.addupdate_scatter(hbuf, [bins + lane_off], w)
                    return 0

                lax.fori_loop(0, nchunk // 8, cbody, 0)

                for h in range(_HBINS // _L):
                    acc = hbuf[pl.ds(h * _L, _L)]
                    for l in range(1, _L):
                        acc = acc + hbuf[pl.ds(l * _HBINS + h * _L, _L)]
                    obuf[pl.ds(q * _HBINS + h * _L, _L)] = acc

            pltpu.async_copy(obuf, out_hbm.at[b], sem_d).wait()

    return hist_kernel(simmat, dtoks)


def _tc_finish(hist2, qtoks):
    BQ = hist2.shape[0]
    q2 = qtoks.reshape(BQ, 1)

    def body(h_ref, q_ref, o_ref):
        qm = (q_ref[...] != 0).astype(jnp.float32)
        o_ref[...] = jnp.log(h_ref[...] * qm + jnp.float32(1e-5))

    return pl.pallas_call(
        body,
        out_shape=jax.ShapeDtypeStruct((BQ, _HBINS), jnp.float32),
    )(hist2, q2)


def kernel(simmat, dtoks, qtoks):
    B, Q, _ = simmat.shape
    hist = _sc_hist(simmat, dtoks.astype(jnp.int32))
    hist2 = hist.reshape(B * Q, _HBINS)
    out = _tc_finish(hist2, qtoks.astype(jnp.int32))
    return out[:, :_BINS].reshape(B, Q, _BINS)


# speedup vs baseline: 90.7272x; 2.7152x over previous
import functools

import jax
import jax.numpy as jnp
from jax import lax
from jax.experimental import pallas as pl
from jax.experimental.pallas import tpu as pltpu
from jax.experimental.pallas import tpu_sc as plsc

_BINS = 30
_HBINS = 32
_L = 16
_NC = 2
_NS = 16
_NW = _NC * _NS


def _sc_hist(simmat, dtoks):
    B, Q, D = simmat.shape
    b_per_w = B // _NW
    nchunk = D // _L

    mesh = plsc.VectorSubcoreMesh(core_axis_name="c", subcore_axis_name="s")

    @functools.partial(
        pl.kernel,
        mesh=mesh,
        compiler_params=pltpu.CompilerParams(needs_layout_passes=False),
        out_type=jax.ShapeDtypeStruct((B, Q * _HBINS), jnp.float32),
        scratch_types=[
            pltpu.VMEM((D,), jnp.int32),
            pltpu.VMEM((D,), jnp.float32),
            pltpu.VMEM((2, D), jnp.float32),
            pltpu.VMEM((_L * _HBINS,), jnp.float32),
            pltpu.VMEM((Q * _HBINS,), jnp.float32),
            pltpu.SemaphoreType.DMA,
            pltpu.SemaphoreType.DMA,
        ],
    )
    def hist_kernel(simmat_hbm, dtoks_hbm, out_hbm,
                    dbuf, wbuf, sbuf, hbuf, obuf, sem_s, sem_d):
        wid = lax.axis_index("s") * _NC + lax.axis_index("c")
        lane_off = lax.iota(jnp.int32, _L) * _HBINS

        for bi in range(b_per_w):
            b = wid * b_per_w + bi

            pltpu.async_copy(dtoks_hbm.at[b], dbuf, sem_d).wait()

            @plsc.parallel_loop(0, nchunk, 1, unroll=8)
            def wbody(i):
                t = dbuf[pl.ds(i * _L, _L)]
                wbuf[pl.ds(i * _L, _L)] = jnp.where(
                    t != 0,
                    jnp.full((_L,), 1.0, jnp.float32),
                    jnp.full((_L,), 0.0, jnp.float32))

            copies = [None, None]
            copies[0] = pltpu.async_copy(simmat_hbm.at[b, 0], sbuf.at[0],
                                         sem_s)
            for q in range(Q):
                cur = q % 2
                copies[cur].wait()
                if q + 1 < Q:
                    copies[1 - cur] = pltpu.async_copy(
                        simmat_hbm.at[b, q + 1], sbuf.at[1 - cur], sem_s)

                for k in range(_HBINS):
                    hbuf[pl.ds(k * _L, _L)] = jnp.zeros((_L,), jnp.float32)

                @plsc.parallel_loop(0, nchunk, 1, unroll=8)
                def cbody(i, cur=cur):
                    v = sbuf[cur, pl.ds(i * _L, _L)]
                    t = (v + jnp.float32(1.000001)) * jnp.float32(0.5)
                    bins = (t * jnp.float32(_BINS - 1)).astype(jnp.int32)
                    w = wbuf[pl.ds(i * _L, _L)]
                    plsc.addupdate_scatter(hbuf, [bins + lane_off], w)

                for h in range(_HBINS // _L):
                    acc = hbuf[pl.ds(h * _L, _L)]
                    for l in range(1, _L):
                        acc = acc + hbuf[pl.ds(l * _HBINS + h * _L, _L)]
                    obuf[pl.ds(q * _HBINS + h * _L, _L)] = acc

            pltpu.async_copy(obuf, out_hbm.at[b], sem_d).wait()

    return hist_kernel(simmat, dtoks)


def _tc_finish(hist2, qtoks):
    BQ = hist2.shape[0]
    q2 = qtoks.reshape(BQ, 1)

    def body(h_ref, q_ref, o_ref):
        qm = (q_ref[...] != 0).astype(jnp.float32)
        o_ref[...] = jnp.log(h_ref[...] * qm + jnp.float32(1e-5))

    return pl.pallas_call(
        body,
        out_shape=jax.ShapeDtypeStruct((BQ, _HBINS), jnp.float32),
    )(hist2, q2)


def kernel(simmat, dtoks, qtoks):
    B, Q, _ = simmat.shape
    hist = _sc_hist(simmat, dtoks.astype(jnp.int32))
    hist2 = hist.reshape(B * Q, _HBINS)
    out = _tc_finish(hist2, qtoks.astype(jnp.int32))
    return out[:, :_BINS].reshape(B, Q, _BINS)


# speedup vs baseline: 107.8450x; 1.1887x over previous
import functools

import jax
import jax.numpy as jnp
from jax import lax
from jax.experimental import pallas as pl
from jax.experimental.pallas import tpu as pltpu
from jax.experimental.pallas import tpu_sc as plsc

_BINS = 30
_HBINS = 32
_L = 16
_NC = 2
_NS = 16
_NW = _NC * _NS
_QG = 4

_LN2 = 0.6931471805599453
_LOG2_COEF = (
    -3.7216296, 10.143928, -15.955576, 19.71584, -17.883608, 11.797779,
    -5.5984416, 1.8633448, -0.41319442, 0.054859888, -0.0033009734,
)


def _vlog(x):
    bits = plsc.bitcast(x, jnp.int32)
    e = ((bits >> 23) - 127).astype(jnp.float32)
    m = plsc.bitcast((bits & 0x007FFFFF) | 0x3F800000, jnp.float32)
    acc = jnp.full((_L,), _LOG2_COEF[10], jnp.float32)
    for k in range(9, -1, -1):
        acc = jnp.float32(_LOG2_COEF[k]) + m * acc
    return (e + acc) * jnp.float32(_LN2)


def _sc_hist(simmat, dtoks, qtoks):
    B, Q, D = simmat.shape
    b_per_w = B // _NW
    nchunk = D // _L
    ngrp = Q // _QG
    hwords = _QG * _L * _HBINS

    mesh = plsc.VectorSubcoreMesh(core_axis_name="c", subcore_axis_name="s")

    @functools.partial(
        pl.kernel,
        mesh=mesh,
        compiler_params=pltpu.CompilerParams(needs_layout_passes=False),
        out_type=jax.ShapeDtypeStruct((B, Q * _BINS), jnp.float32),
        scratch_types=[
            pltpu.VMEM((D,), jnp.int32),
            pltpu.VMEM((D,), jnp.float32),
            pltpu.VMEM((2, _QG, D), jnp.float32),
            pltpu.VMEM((hwords,), jnp.float32),
            pltpu.VMEM((Q * _BINS,), jnp.float32),
            pltpu.VMEM((_L,), jnp.int32),
            pltpu.SemaphoreType.DMA,
            pltpu.SemaphoreType.DMA,
        ],
    )
    def hist_kernel(simmat_hbm, dtoks_hbm, qtoks_hbm, out_hbm,
                    dbuf, wbuf, sbuf, hbuf, obuf, qbuf, sem_s, sem_d):
        wid = lax.axis_index("s") * _NC + lax.axis_index("c")
        lane = lax.iota(jnp.int32, _L)
        row_off = [lane * _HBINS + jnp.int32(r * _L * _HBINS)
                   for r in range(_QG)]
        ones = jnp.full((_L,), 1.0, jnp.float32)
        zeros = jnp.full((_L,), 0.0, jnp.float32)

        for bi in range(b_per_w):
            b = wid * b_per_w + bi

            pltpu.async_copy(dtoks_hbm.at[b], dbuf, sem_d).wait()
            pltpu.async_copy(qtoks_hbm.at[b], qbuf, sem_d).wait()

            @plsc.parallel_loop(0, nchunk, 1, unroll=8)
            def wbody(i):
                t = dbuf[pl.ds(i * _L, _L)]
                wbuf[pl.ds(i * _L, _L)] = jnp.where(t != 0, ones, zeros)

            copies = [None, None]
            copies[0] = pltpu.async_copy(
                simmat_hbm.at[b, pl.ds(0, _QG)], sbuf.at[0], sem_s)
            for g in range(ngrp):
                cur = g % 2
                copies[cur].wait()
                if g + 1 < ngrp:
                    copies[1 - cur] = pltpu.async_copy(
                        simmat_hbm.at[b, pl.ds((g + 1) * _QG, _QG)],
                        sbuf.at[1 - cur], sem_s)

                @plsc.parallel_loop(0, hwords // _L, 1, unroll=8)
                def zbody(i):
                    hbuf[pl.ds(i * _L, _L)] = zeros

                @plsc.parallel_loop(0, nchunk, 1, unroll=2)
                def cbody(i, cur=cur):
                    w = wbuf[pl.ds(i * _L, _L)]
                    for r in range(_QG):
                        v = sbuf[cur, r, pl.ds(i * _L, _L)]
                        t = (v + jnp.float32(1.000001)) * jnp.float32(0.5)
                        bins = (t * jnp.float32(_BINS - 1)).astype(jnp.int32)
                        plsc.addupdate_scatter(hbuf, [bins + row_off[r]], w)

                for r in range(_QG):
                    q = g * _QG + r
                    qmv = plsc.load_gather(
                        qbuf, [jnp.full((_L,), q, jnp.int32)])
                    qm = jnp.where(qmv != 0, ones, zeros)
                    rbase = r * _L * _HBINS
                    for h in range(2):
                        hb = h * (_BINS - _L)
                        acc = hbuf[pl.ds(rbase + hb, _L)]
                        for l in range(1, _L):
                            acc = acc + hbuf[pl.ds(rbase + l * _HBINS + hb,
                                                   _L)]
                        res = _vlog(acc * qm + jnp.float32(1e-5))
                        obuf[pl.ds(q * _BINS + hb, _L)] = res

            pltpu.async_copy(obuf, out_hbm.at[b], sem_d).wait()

    return hist_kernel(simmat, dtoks, qtoks)


def kernel(simmat, dtoks, qtoks):
    B, Q, _ = simmat.shape
    out = _sc_hist(simmat, dtoks.astype(jnp.int32), qtoks.astype(jnp.int32))
    return out.reshape(B, Q, _BINS)
